# baseline (speedup 1.0000x reference)
import functools

import jax
import jax.numpy as jnp
from jax import lax
from jax.experimental import pallas as pl
from jax.experimental.pallas import tpu as pltpu
from jax.experimental.pallas import tpu_sc as plsc

_N = 10000
_E = 160000
_IN = 128
_HID = 4
_EDGE = 16
_OUT = 128
_K = _HID * _IN

_NC = 2
_NS = 16
_NW = _NC * _NS
_CH = 128
_NCH = 40
_EPW = _NCH * _CH
_EPAD = _NW * _EPW
_NPAD = 10240
_HW = 8
_RPT = _NPAD // _NS

def _gather_body(x_hbm, idx_hbm, out_hbm, idx_v, rows0, rows1, sem0, sem1):
    c = lax.axis_index("c")
    s = lax.axis_index("s")
    wid = c * _NS + s
    base = wid * _EPW
    pltpu.sync_copy(idx_hbm.at[wid], idx_v)
    bufs = (rows0, rows1)
    sems = (sem0, sem1)
    descs = [None, None]
    descs[0] = pltpu.async_copy(x_hbm.at[idx_v.at[0]], rows0, sem0)
    for j in range(1, _NCH):
        b = j % 2
        descs[b] = pltpu.async_copy(x_hbm.at[idx_v.at[j]], bufs[b], sems[b])
        descs[1 - b].wait()
        pltpu.sync_copy(bufs[1 - b],
                        out_hbm.at[pl.ds(base + (j - 1) * _CH, _CH)])
    last = (_NCH - 1) % 2
    descs[last].wait()
    pltpu.sync_copy(bufs[last],
                    out_hbm.at[pl.ds(base + (_NCH - 1) * _CH, _CH)])


def _scatter_body(msgs_hbm, idx_hbm, zeros_hbm, out_hbm, msgs_v, idx_v, agg_sh):
    c = lax.axis_index("c")
    s = lax.axis_index("s")
    wid = c * _NS + s
    pltpu.sync_copy(zeros_hbm.at[pl.ds(s * _RPT, _RPT)],
                    agg_sh.at[pl.ds(s * _RPT, _RPT)])
    plsc.subcore_barrier()
    for j in range(_NCH):
        pltpu.sync_copy(idx_hbm.at[wid, j], idx_v)
        pltpu.sync_copy(msgs_hbm.at[wid, j], msgs_v)
        pltpu.sync_copy(msgs_v, agg_sh.at[idx_v], add=True)
    plsc.subcore_barrier()
    pltpu.sync_copy(agg_sh.at[pl.ds(s * _RPT, _RPT)],
                    out_hbm.at[c, pl.ds(s * _RPT, _RPT)])


@functools.lru_cache(maxsize=None)
def _sc_kernels():
    mesh = plsc.VectorSubcoreMesh(core_axis_name="c", subcore_axis_name="s")
    params = pltpu.CompilerParams(use_tc_tiling_on_sc=False)
    gather = pl.kernel(
        _gather_body,
        out_type=jax.ShapeDtypeStruct((_EPAD, _IN), jnp.bfloat16),
        mesh=mesh,
        compiler_params=params,
        scratch_types=[
            pltpu.VMEM((_NCH, _CH), jnp.int32),
            pltpu.VMEM((_CH, _IN), jnp.bfloat16),
            pltpu.VMEM((_CH, _IN), jnp.bfloat16),
            pltpu.SemaphoreType.DMA,
            pltpu.SemaphoreType.DMA,
        ],
    )
    scatter = pl.kernel(
        _scatter_body,
        out_type=jax.ShapeDtypeStruct((_NC, _NPAD, _HW), jnp.float32),
        mesh=mesh,
        compiler_params=params,
        scratch_types=[
            pltpu.VMEM((_CH, _HW), jnp.float32),
            pltpu.VMEM((_CH,), jnp.int32),
            pltpu.VMEM_SHARED((_NPAD, _HW), jnp.float32),
        ],
    )
    return gather, scatter


_EB = 2048


def _msgs_body(ea_ref, xj_ref, w1_ref, b1_ref, w2_ref, b2_ref, out_ref):
    h = jnp.maximum(
        jnp.dot(ea_ref[:], w1_ref[:], preferred_element_type=jnp.float32)
        + b1_ref[:], 0.0)
    we = jnp.dot(h, w2_ref[:], preferred_element_type=jnp.float32) + b2_ref[:]
    xj = xj_ref[:]
    parts = [
        jnp.sum(we[:, o * _IN:(o + 1) * _IN] * xj, axis=1, keepdims=True)
        for o in range(_HID)
    ]
    parts.append(jnp.zeros((ea_ref.shape[0], _HW - _HID), jnp.float32))
    out_ref[:] = jnp.concatenate(parts, axis=1)


def _tc_msgs(ea, xj, W1, b1, W2q, b2q):
    return pl.pallas_call(
        _msgs_body,
        grid=(_EPAD // _EB,),
        in_specs=[
            pl.BlockSpec((_EB, _EDGE), lambda i: (i, 0)),
            pl.BlockSpec((_EB, _IN), lambda i: (i, 0)),
            pl.BlockSpec((_EDGE, _K), lambda i: (0, 0)),
            pl.BlockSpec((1, _K), lambda i: (0, 0)),
            pl.BlockSpec((_K, _K), lambda i: (0, 0)),
            pl.BlockSpec((1, _K), lambda i: (0, 0)),
        ],
        out_specs=pl.BlockSpec((_EB, _HW), lambda i: (i, 0)),
        out_shape=jax.ShapeDtypeStruct((_EPAD, _HW), jnp.float32),
    )(ea, xj, W1, b1, W2q, b2q)


_NB = 2000


def _final_body(agg_ref, x_ref, rw_ref, cb_ref, dw_ref, db_ref, out_ref):
    a = (agg_ref[0] + agg_ref[1])[:, :_HID]
    enc = jnp.maximum(
        a + jnp.dot(x_ref[:], rw_ref[:], preferred_element_type=jnp.float32)
        + cb_ref[:], 0.0)
    out_ref[:] = (
        jnp.dot(enc, dw_ref[:], preferred_element_type=jnp.float32) + db_ref[:])


def _tc_final(agg, x, root_W, conv_bias, dec_W, dec_b):
    return pl.pallas_call(
        _final_body,
        grid=(_N // _NB,),
        in_specs=[
            pl.BlockSpec((_NC, _NB, _HW), lambda i: (0, i, 0)),
            pl.BlockSpec((_NB, _IN), lambda i: (i, 0)),
            pl.BlockSpec((_IN, _HID), lambda i: (0, 0)),
            pl.BlockSpec((1, _HID), lambda i: (0, 0)),
            pl.BlockSpec((_HID, _OUT), lambda i: (0, 0)),
            pl.BlockSpec((1, _OUT), lambda i: (0, 0)),
        ],
        out_specs=pl.BlockSpec((_NB, _OUT), lambda i: (i, 0)),
        out_shape=jax.ShapeDtypeStruct((_N, _OUT), jnp.float32),
    )(agg, x, root_W, conv_bias, dec_W, dec_b)


def kernel(x, edge_index, edge_attr, W1, b1, W2, b2, root_W, conv_bias,
           dec_W, dec_b):
    pad = _EPAD - _E
    src = jnp.concatenate(
        [edge_index[0], jnp.zeros((pad,), jnp.int32)]).reshape(_NW, _NCH, _CH)
    dst = jnp.concatenate(
        [edge_index[1],
         jnp.full((pad,), _NPAD - 1, jnp.int32)]).reshape(_NW, _NCH, _CH)
    W2q = W2.reshape(_K, _IN, _HID).transpose(0, 2, 1).reshape(_K, _K)
    b2q = b2.reshape(_IN, _HID).T.reshape(1, _K)

    sc_gather, sc_scatter = _sc_kernels()
    x_j = sc_gather(x.astype(jnp.bfloat16), src).astype(jnp.float32)
    ea_pad = jnp.concatenate(
        [edge_attr, jnp.zeros((pad, _EDGE), jnp.float32)], 0)
    msgs = _tc_msgs(ea_pad, x_j, W1, b1.reshape(1, _K), W2q, b2q)
    agg2 = sc_scatter(msgs.reshape(_NW, _NCH, _CH, _HW), dst,
                      jnp.zeros((_NPAD, _HW), jnp.float32))
    out = _tc_final(agg2[:, :_N, :], x, root_W, conv_bias.reshape(1, _HID),
                    dec_W, dec_b.reshape(1, _OUT))
    return out


# speedup vs baseline: 1.0899x; 1.0899x over previous
import functools

import jax
import jax.numpy as jnp
from jax import lax
from jax.experimental import pallas as pl
from jax.experimental.pallas import tpu as pltpu
from jax.experimental.pallas import tpu_sc as plsc

_N = 10000
_E = 160000
_IN = 128
_HID = 4
_EDGE = 16
_OUT = 128
_K = _HID * _IN

_NC = 2
_NS = 16
_NW = _NC * _NS
_CH = 128
_NCH = 40
_EPW = _NCH * _CH
_EPAD = _NW * _EPW
_NPAD = 10240
_HW = 8
_RPT = _NPAD // _NS
_S = 4
_NCHS = _NCH // _S
_EPS = _EPAD // _S

def _gather_body(x_hbm, idx_hbm, out_hbm, idx_v, rows0, rows1, sem0, sem1):
    c = lax.axis_index("c")
    s = lax.axis_index("s")
    wid = c * _NS + s
    base = wid * (_NCHS * _CH)
    pltpu.sync_copy(idx_hbm.at[wid], idx_v)
    bufs = (rows0, rows1)
    sems = (sem0, sem1)
    descs = [None, None]
    descs[0] = pltpu.async_copy(x_hbm.at[idx_v.at[0]], rows0, sem0)
    for j in range(1, _NCHS):
        b = j % 2
        descs[b] = pltpu.async_copy(x_hbm.at[idx_v.at[j]], bufs[b], sems[b])
        descs[1 - b].wait()
        pltpu.sync_copy(bufs[1 - b],
                        out_hbm.at[pl.ds(base + (j - 1) * _CH, _CH)])
    last = (_NCHS - 1) % 2
    descs[last].wait()
    pltpu.sync_copy(bufs[last],
                    out_hbm.at[pl.ds(base + (_NCHS - 1) * _CH, _CH)])


def _scatter_body(msgs_hbm, idx_hbm, zeros_hbm, out_hbm, msgs_v, idx_v, agg_sh):
    c = lax.axis_index("c")
    s = lax.axis_index("s")
    wid = c * _NS + s
    pltpu.sync_copy(zeros_hbm.at[pl.ds(s * _RPT, _RPT)],
                    agg_sh.at[pl.ds(s * _RPT, _RPT)])
    plsc.subcore_barrier()
    for j in range(_NCH):
        pltpu.sync_copy(idx_hbm.at[wid, j], idx_v)
        pltpu.sync_copy(msgs_hbm.at[wid, j], msgs_v)
        pltpu.sync_copy(msgs_v, agg_sh.at[idx_v], add=True)
    plsc.subcore_barrier()
    pltpu.sync_copy(agg_sh.at[pl.ds(s * _RPT, _RPT)],
                    out_hbm.at[c, pl.ds(s * _RPT, _RPT)])


@functools.lru_cache(maxsize=None)
def _sc_kernels():
    mesh = plsc.VectorSubcoreMesh(core_axis_name="c", subcore_axis_name="s")
    params = pltpu.CompilerParams(use_tc_tiling_on_sc=False)
    gather = pl.kernel(
        _gather_body,
        out_type=jax.ShapeDtypeStruct((_EPS, _IN), jnp.float32),
        mesh=mesh,
        compiler_params=params,
        scratch_types=[
            pltpu.VMEM((_NCHS, _CH), jnp.int32),
            pltpu.VMEM((_CH, _IN), jnp.float32),
            pltpu.VMEM((_CH, _IN), jnp.float32),
            pltpu.SemaphoreType.DMA,
            pltpu.SemaphoreType.DMA,
        ],
    )
    scatter = pl.kernel(
        _scatter_body,
        out_type=jax.ShapeDtypeStruct((_NC, _NPAD, _HW), jnp.float32),
        mesh=mesh,
        compiler_params=params,
        scratch_types=[
            pltpu.VMEM((_CH, _HW), jnp.float32),
            pltpu.VMEM((_CH,), jnp.int32),
            pltpu.VMEM_SHARED((_NPAD, _HW), jnp.float32),
        ],
    )
    return gather, scatter


_EB = 2048


def _msgs_body(ea_ref, xj_ref, w1_ref, b1_ref, w2_ref, b2_ref, out_ref):
    h = jnp.maximum(
        jnp.dot(ea_ref[:], w1_ref[:], preferred_element_type=jnp.float32)
        + b1_ref[:], 0.0)
    we = jnp.dot(h, w2_ref[:], preferred_element_type=jnp.float32) + b2_ref[:]
    xj = xj_ref[:]
    parts = [
        jnp.sum(we[:, o * _IN:(o + 1) * _IN] * xj, axis=1, keepdims=True)
        for o in range(_HID)
    ]
    parts.append(jnp.zeros((ea_ref.shape[0], _HW - _HID), jnp.float32))
    out_ref[:] = jnp.concatenate(parts, axis=1)


def _tc_msgs(ea, xj, W1, b1, W2q, b2q):
    return pl.pallas_call(
        _msgs_body,
        grid=(_EPS // _EB,),
        in_specs=[
            pl.BlockSpec((_EB, _EDGE), lambda i: (i, 0)),
            pl.BlockSpec((_EB, _IN), lambda i: (i, 0)),
            pl.BlockSpec((_EDGE, _K), lambda i: (0, 0)),
            pl.BlockSpec((1, _K), lambda i: (0, 0)),
            pl.BlockSpec((_K, _K), lambda i: (0, 0)),
            pl.BlockSpec((1, _K), lambda i: (0, 0)),
        ],
        out_specs=pl.BlockSpec((_EB, _HW), lambda i: (i, 0)),
        out_shape=jax.ShapeDtypeStruct((_EPS, _HW), jnp.float32),
    )(ea, xj, W1, b1, W2q, b2q)


_NB = 2000


def _final_body(agg_ref, x_ref, rw_ref, cb_ref, dw_ref, db_ref, out_ref):
    a = (agg_ref[0] + agg_ref[1])[:, :_HID]
    enc = jnp.maximum(
        a + jnp.dot(x_ref[:], rw_ref[:], preferred_element_type=jnp.float32)
        + cb_ref[:], 0.0)
    out_ref[:] = (
        jnp.dot(enc, dw_ref[:], preferred_element_type=jnp.float32) + db_ref[:])


def _tc_final(agg, x, root_W, conv_bias, dec_W, dec_b):
    return pl.pallas_call(
        _final_body,
        grid=(_N // _NB,),
        in_specs=[
            pl.BlockSpec((_NC, _NB, _HW), lambda i: (0, i, 0)),
            pl.BlockSpec((_NB, _IN), lambda i: (i, 0)),
            pl.BlockSpec((_IN, _HID), lambda i: (0, 0)),
            pl.BlockSpec((1, _HID), lambda i: (0, 0)),
            pl.BlockSpec((_HID, _OUT), lambda i: (0, 0)),
            pl.BlockSpec((1, _OUT), lambda i: (0, 0)),
        ],
        out_specs=pl.BlockSpec((_NB, _OUT), lambda i: (i, 0)),
        out_shape=jax.ShapeDtypeStruct((_N, _OUT), jnp.float32),
    )(agg, x, root_W, conv_bias, dec_W, dec_b)


def kernel(x, edge_index, edge_attr, W1, b1, W2, b2, root_W, conv_bias,
           dec_W, dec_b):
    pad = _EPAD - _E
    src = jnp.concatenate([edge_index[0], jnp.zeros((pad,), jnp.int32)])
    dst = jnp.concatenate(
        [edge_index[1],
         jnp.full((pad,), _NPAD - 1, jnp.int32)]).reshape(_NW, _NCH, _CH)
    W2q = W2.reshape(_K, _IN, _HID).transpose(0, 2, 1).reshape(_K, _K)
    b2q = b2.reshape(_IN, _HID).T.reshape(1, _K)

    sc_gather, sc_scatter = _sc_kernels()
    ea_pad = jnp.concatenate(
        [edge_attr, jnp.zeros((pad, _EDGE), jnp.float32)], 0)
    src_s = src.reshape(_S, _NW, _NCHS, _CH)
    b1r, b2r = b1.reshape(1, _K), b2q
    msgs_parts = []
    for si in range(_S):
        xj_s = sc_gather(x, src_s[si])
        msgs_parts.append(
            _tc_msgs(ea_pad[si * _EPS:(si + 1) * _EPS], xj_s, W1, b1r,
                     W2q, b2r))
    msgs = jnp.concatenate(msgs_parts, 0)
    agg2 = sc_scatter(msgs.reshape(_NW, _NCH, _CH, _HW), dst,
                      jnp.zeros((_NPAD, _HW), jnp.float32))
    out = _tc_final(agg2[:, :_N, :], x, root_W, conv_bias.reshape(1, _HID),
                    dec_W, dec_b.reshape(1, _OUT))
    return out


# speedup vs baseline: 1.1251x; 1.0323x over previous
import functools

import jax
import jax.numpy as jnp
from jax import lax
from jax.experimental import pallas as pl
from jax.experimental.pallas import tpu as pltpu
from jax.experimental.pallas import tpu_sc as plsc

_N = 10000
_E = 160000
_IN = 128
_HID = 4
_EDGE = 16
_OUT = 128
_K = _HID * _IN

_NC = 2
_NS = 16
_NW = _NC * _NS
_CH = 128
_NCH = 40
_EPW = _NCH * _CH
_EPAD = _NW * _EPW
_NPAD = 10240
_HW = 8
_RPT = _NPAD // _NS
_S = 1
_NCHS = _NCH // _S
_EPS = _EPAD // _S

def _gather_body(x_hbm, idx_hbm, out_hbm, idx_v, rows0, rows1, sem0, sem1):
    c = lax.axis_index("c")
    s = lax.axis_index("s")
    wid = c * _NS + s
    base = wid * (_NCHS * _CH)
    pltpu.sync_copy(idx_hbm.at[wid], idx_v)
    bufs = (rows0, rows1)
    sems = (sem0, sem1)
    descs = [None, None]
    descs[0] = pltpu.async_copy(x_hbm.at[idx_v.at[0]], rows0, sem0)
    for j in range(1, _NCHS):
        b = j % 2
        descs[b] = pltpu.async_copy(x_hbm.at[idx_v.at[j]], bufs[b], sems[b])
        descs[1 - b].wait()
        pltpu.sync_copy(bufs[1 - b],
                        out_hbm.at[pl.ds(base + (j - 1) * _CH, _CH)])
    last = (_NCHS - 1) % 2
    descs[last].wait()
    pltpu.sync_copy(bufs[last],
                    out_hbm.at[pl.ds(base + (_NCHS - 1) * _CH, _CH)])


def _scatter_body(msgs_hbm, idx_hbm, zeros_hbm, out_hbm, msgs_v, idx_v, agg_sh):
    c = lax.axis_index("c")
    s = lax.axis_index("s")
    wid = c * _NS + s
    pltpu.sync_copy(zeros_hbm.at[pl.ds(s * _RPT, _RPT)],
                    agg_sh.at[pl.ds(s * _RPT, _RPT)])
    plsc.subcore_barrier()
    for j in range(_NCH):
        pltpu.sync_copy(idx_hbm.at[wid, j], idx_v)
        pltpu.sync_copy(msgs_hbm.at[wid, j], msgs_v)
        pltpu.sync_copy(msgs_v, agg_sh.at[idx_v], add=True)
    plsc.subcore_barrier()
    pltpu.sync_copy(agg_sh.at[pl.ds(s * _RPT, _RPT)],
                    out_hbm.at[c, pl.ds(s * _RPT, _RPT)])


@functools.lru_cache(maxsize=None)
def _sc_kernels():
    mesh = plsc.VectorSubcoreMesh(core_axis_name="c", subcore_axis_name="s")
    params = pltpu.CompilerParams(use_tc_tiling_on_sc=False)
    gather = pl.kernel(
        _gather_body,
        out_type=jax.ShapeDtypeStruct((_EPS, _IN), jnp.float32),
        mesh=mesh,
        compiler_params=params,
        scratch_types=[
            pltpu.VMEM((_NCHS, _CH), jnp.int32),
            pltpu.VMEM((_CH, _IN), jnp.float32),
            pltpu.VMEM((_CH, _IN), jnp.float32),
            pltpu.SemaphoreType.DMA,
            pltpu.SemaphoreType.DMA,
        ],
    )
    scatter = pl.kernel(
        _scatter_body,
        out_type=jax.ShapeDtypeStruct((_NC, _NPAD, _HW), jnp.float32),
        mesh=mesh,
        compiler_params=params,
        scratch_types=[
            pltpu.VMEM((_CH, _HW), jnp.float32),
            pltpu.VMEM((_CH,), jnp.int32),
            pltpu.VMEM_SHARED((_NPAD, _HW), jnp.float32),
        ],
    )
    return gather, scatter


_EB = 2048


def _msgs_body(ea_ref, xj_ref, w1_ref, b1_ref, w2_ref, b2_ref, out_ref):
    h = jnp.maximum(
        jnp.dot(ea_ref[:], w1_ref[:], preferred_element_type=jnp.float32)
        + b1_ref[:], 0.0)
    we = jnp.dot(h, w2_ref[:], preferred_element_type=jnp.float32) + b2_ref[:]
    xj = xj_ref[:]
    parts = [
        jnp.sum(we[:, o * _IN:(o + 1) * _IN] * xj, axis=1, keepdims=True)
        for o in range(_HID)
    ]
    parts.append(jnp.zeros((ea_ref.shape[0], _HW - _HID), jnp.float32))
    out_ref[:] = jnp.concatenate(parts, axis=1)


def _tc_msgs(ea, xj, W1, b1, W2q, b2q):
    return pl.pallas_call(
        _msgs_body,
        grid=(_EPS // _EB,),
        in_specs=[
            pl.BlockSpec((_EB, _EDGE), lambda i: (i, 0)),
            pl.BlockSpec((_EB, _IN), lambda i: (i, 0)),
            pl.BlockSpec((_EDGE, _K), lambda i: (0, 0)),
            pl.BlockSpec((1, _K), lambda i: (0, 0)),
            pl.BlockSpec((_K, _K), lambda i: (0, 0)),
            pl.BlockSpec((1, _K), lambda i: (0, 0)),
        ],
        out_specs=pl.BlockSpec((_EB, _HW), lambda i: (i, 0)),
        out_shape=jax.ShapeDtypeStruct((_EPS, _HW), jnp.float32),
    )(ea, xj, W1, b1, W2q, b2q)


_NB = 2000


def _final_body(agg_ref, x_ref, rw_ref, cb_ref, dw_ref, db_ref, out_ref):
    a = (agg_ref[0] + agg_ref[1])[:, :_HID]
    enc = jnp.maximum(
        a + jnp.dot(x_ref[:], rw_ref[:], preferred_element_type=jnp.float32)
        + cb_ref[:], 0.0)
    out_ref[:] = (
        jnp.dot(enc, dw_ref[:], preferred_element_type=jnp.float32) + db_ref[:])


def _tc_final(agg, x, root_W, conv_bias, dec_W, dec_b):
    return pl.pallas_call(
        _final_body,
        grid=(_N // _NB,),
        in_specs=[
            pl.BlockSpec((_NC, _NB, _HW), lambda i: (0, i, 0)),
            pl.BlockSpec((_NB, _IN), lambda i: (i, 0)),
            pl.BlockSpec((_IN, _HID), lambda i: (0, 0)),
            pl.BlockSpec((1, _HID), lambda i: (0, 0)),
            pl.BlockSpec((_HID, _OUT), lambda i: (0, 0)),
            pl.BlockSpec((1, _OUT), lambda i: (0, 0)),
        ],
        out_specs=pl.BlockSpec((_NB, _OUT), lambda i: (i, 0)),
        out_shape=jax.ShapeDtypeStruct((_N, _OUT), jnp.float32),
    )(agg, x, root_W, conv_bias, dec_W, dec_b)


def kernel(x, edge_index, edge_attr, W1, b1, W2, b2, root_W, conv_bias,
           dec_W, dec_b):
    pad = _EPAD - _E
    src = jnp.concatenate([edge_index[0], jnp.zeros((pad,), jnp.int32)])
    dst = jnp.concatenate(
        [edge_index[1],
         jnp.full((pad,), _NPAD - 1, jnp.int32)]).reshape(_NW, _NCH, _CH)
    W2q = W2.reshape(_K, _IN, _HID).transpose(0, 2, 1).reshape(_K, _K)
    b2q = b2.reshape(_IN, _HID).T.reshape(1, _K)

    sc_gather, sc_scatter = _sc_kernels()
    ea_pad = jnp.concatenate(
        [edge_attr, jnp.zeros((pad, _EDGE), jnp.float32)], 0)
    src_s = src.reshape(_S, _NW, _NCHS, _CH)
    b1r, b2r = b1.reshape(1, _K), b2q
    msgs_parts = []
    for si in range(_S):
        xj_s = sc_gather(x, src_s[si])
        msgs_parts.append(
            _tc_msgs(ea_pad[si * _EPS:(si + 1) * _EPS], xj_s, W1, b1r,
                     W2q, b2r))
    msgs = jnp.concatenate(msgs_parts, 0)
    agg2 = sc_scatter(msgs.reshape(_NW, _NCH, _CH, _HW), dst,
                      jnp.zeros((_NPAD, _HW), jnp.float32))
    out = _tc_final(agg2[:, :_N, :], x, root_W, conv_bias.reshape(1, _HID),
                    dec_W, dec_b.reshape(1, _OUT))
    return out
